# baseline (speedup 1.0000x reference)
import jax
import jax.numpy as jnp
from jax.experimental import pallas as pl
from jax.experimental.pallas import tpu as pltpu

B, P, D, H = 32, 64, 2, 64
G = 4


def _body(dq1_ref, dq2_ref, dp1_ref, dp2_ref, m_ref, l_ref, k_ref,
          wn_ref, bn_ref, we_ref, be_ref, wmsg_ref, bm_ref,
          wupd_ref, bu_ref, wd_ref, bd_ref, out0_ref, out1_ref):
    f32 = jnp.float32
    dot = lambda x, w: jnp.dot(x, w, preferred_element_type=f32)
    n = G * P
    dq1 = dq1_ref[...].reshape(n, D)
    dq2 = dq2_ref[...].reshape(n, D)
    dp1 = dp1_ref[...].reshape(n, D)
    dp2 = dp2_ref[...].reshape(n, D)
    mr = m_ref[...].reshape(n, 1)
    wn = wn_ref[...]
    bn = bn_ref[...]
    wme = wmsg_ref[0:H, :]
    wms = wmsg_ref[H:2 * H, :]
    wmr = wmsg_ref[2 * H:3 * H, :]
    wu1 = wupd_ref[0:H, :]
    wu2 = wupd_ref[H:2 * H, :]
    hn, a_t, c_t = [], [], []
    for d in range(D):
        na_d = jnp.concatenate(
            [dq1[:, d:d + 1], dq2[:, d:d + 1], dp1[:, d:d + 1],
             dp2[:, d:d + 1], mr], axis=1)
        hn_d = jax.nn.relu(dot(na_d, wn) + bn)
        hn.append(hn_d)
        a_t.append(dot(hn_d, wms))
        c_t.append(dot(hn_d, wmr))
    a_t = jnp.concatenate(a_t, axis=1)
    c_t = jnp.concatenate(c_t, axis=1)
    ea = jnp.concatenate([l_ref[...].reshape(G * P * P, 1),
                          k_ref[...].reshape(G * P * P, 1)], axis=1)
    he = jax.nn.relu(dot(ea, we_ref[...]) + be_ref[...])
    wme2 = jnp.concatenate([wme, wme], axis=1)
    eh = dot(he, wme2).reshape(G, P, P, D * H)
    bm2 = jnp.concatenate([bm_ref[...], bm_ref[...]], axis=1)
    t = jax.nn.relu(eh + a_t.reshape(G, 1, P, D * H)
                    + c_t.reshape(G, P, 1, D * H) + bm2)
    agg = jnp.sum(t, axis=2).reshape(n, D * H)
    outs = []
    for d in range(D):
        h2_d = jax.nn.relu(dot(hn[d], wu1)
                           + dot(agg[:, d * H:(d + 1) * H], wu2) + bu_ref[...])
        outs.append(dot(h2_d, wd_ref[...]) + bd_ref[...])
    out0_ref[...] = jnp.concatenate(
        [outs[0][:, 0:1], outs[1][:, 0:1]], axis=1).reshape(G, P, D)
    out1_ref[...] = jnp.concatenate(
        [outs[0][:, 1:2], outs[1][:, 1:2]], axis=1).reshape(G, P, D)


def kernel(dq1, dq2, dp1, dp2, m, t, dt, length, k,
           Wn_enc, bn_enc, We_enc, be_enc, W_msg, b_msg, W_upd, b_upd,
           W_dec, b_dec):
    del t, dt
    per_b = lambda shape: pl.BlockSpec(shape, lambda b: (b, 0, 0))
    const = lambda shape: pl.BlockSpec(shape, lambda b: (0, 0))

    out0, out1 = pl.pallas_call(
        _body,
        grid=(B // G,),
        in_specs=[
            per_b((G, P, D)),
            per_b((G, P, D)),
            per_b((G, P, D)),
            per_b((G, P, D)),
            per_b((G, P, 1)),
            per_b((G, P * P, 1)),
            per_b((G, P * P, 1)),
            const((5, H)),
            const((1, H)),
            const((2, H)),
            const((1, H)),
            const((3 * H, H)),
            const((1, H)),
            const((2 * H, H)),
            const((1, H)),
            const((H, 2)),
            const((1, 2)),
        ],
        out_specs=[per_b((G, P, D)), per_b((G, P, D))],
        out_shape=[jax.ShapeDtypeStruct((B, P, D), jnp.float32),
                   jax.ShapeDtypeStruct((B, P, D), jnp.float32)],
        compiler_params=pltpu.CompilerParams(
            dimension_semantics=("arbitrary",)),
    )(dq1, dq2, dp1, dp2, m,
      length.reshape(B, P * P, 1), k.reshape(B, P * P, 1),
      Wn_enc, bn_enc.reshape(1, H), We_enc, be_enc.reshape(1, H),
      W_msg, b_msg.reshape(1, H), W_upd, b_upd.reshape(1, H),
      W_dec, b_dec.reshape(1, 2))
    return out0, out1


# speedup vs baseline: 2.2993x; 2.2993x over previous
import jax
import jax.numpy as jnp
from jax.experimental import pallas as pl
from jax.experimental.pallas import tpu as pltpu

B, P, D, H = 32, 64, 2, 64
G = 4


def _body(na_ref, ea_ref, wn_ref, bn_ref, we_ref, be_ref,
          wme_ref, wms_ref, wmr_ref, bm_ref, wu1_ref, wu2_ref, bu_ref,
          wd_ref, bd_ref, out_ref):
    f32 = jnp.float32
    na = na_ref[...].reshape(G * P, D * 5)
    hn = jax.nn.relu(jnp.dot(na, wn_ref[...],
                             preferred_element_type=f32) + bn_ref[...])
    a_term = jnp.dot(hn, wms_ref[...], preferred_element_type=f32)
    c_term = jnp.dot(hn, wmr_ref[...], preferred_element_type=f32)
    he = jax.nn.relu(jnp.dot(ea_ref[...].reshape(G * P * P, 2), we_ref[...],
                             preferred_element_type=f32) + be_ref[...])
    eh = jnp.dot(he, wme_ref[...], preferred_element_type=f32)
    eh4 = eh.reshape(G, P, P, D * H)
    t = jax.nn.relu(eh4 + a_term.reshape(G, 1, P, D * H)
                    + c_term.reshape(G, P, 1, D * H) + bm_ref[...])
    agg = jnp.sum(t, axis=2).reshape(G * P, D * H)
    h2 = jax.nn.relu(jnp.dot(hn, wu1_ref[...], preferred_element_type=f32)
                     + jnp.dot(agg, wu2_ref[...], preferred_element_type=f32)
                     + bu_ref[...])
    o = jnp.dot(h2, wd_ref[...], preferred_element_type=f32) + bd_ref[...]
    out_ref[...] = o.reshape(G, P, D * 2)


def _blkdiag(w):
    r, c = w.shape
    z = jnp.zeros((r, c), w.dtype)
    return jnp.concatenate(
        [jnp.concatenate([w, z], axis=1), jnp.concatenate([z, w], axis=1)],
        axis=0)


def kernel(dq1, dq2, dp1, dp2, m, t, dt, length, k,
           Wn_enc, bn_enc, We_enc, be_enc, W_msg, b_msg, W_upd, b_upd,
           W_dec, b_dec):
    del t, dt
    m_rep = jnp.tile(m, (1, 1, D))
    na = jnp.stack([dq1, dq2, dp1, dp2, m_rep], axis=-1).reshape(B, P, D * 5)
    sel0 = jnp.array([1.0, 0.0], jnp.float32)
    sel1 = jnp.array([0.0, 1.0], jnp.float32)
    ea = (length.reshape(B, P * P, 1) * sel0
          + k.reshape(B, P * P, 1) * sel1)
    wme, wms, wmr = W_msg[:H], W_msg[H:2 * H], W_msg[2 * H:]
    wu1, wu2 = W_upd[:H], W_upd[H:]
    two = lambda v: jnp.concatenate([v, v]).reshape(1, -1)

    per_b3 = lambda shape: pl.BlockSpec(shape, lambda b: (b, 0, 0))
    const2 = lambda shape: pl.BlockSpec(shape, lambda b: (0, 0))

    out = pl.pallas_call(
        _body,
        grid=(B // G,),
        in_specs=[
            per_b3((G, P, D * 5)),
            per_b3((G, P * P, 2)),
            const2((D * 5, D * H)),
            const2((1, D * H)),
            const2((2, H)),
            const2((1, H)),
            const2((H, D * H)),
            const2((D * H, D * H)),
            const2((D * H, D * H)),
            const2((1, D * H)),
            const2((D * H, D * H)),
            const2((D * H, D * H)),
            const2((1, D * H)),
            const2((D * H, D * 2)),
            const2((1, D * 2)),
        ],
        out_specs=per_b3((G, P, D * 2)),
        out_shape=jax.ShapeDtypeStruct((B, P, D * 2), jnp.float32),
        compiler_params=pltpu.CompilerParams(
            dimension_semantics=("arbitrary",)),
    )(na, ea, _blkdiag(Wn_enc), two(bn_enc), We_enc, be_enc.reshape(1, -1),
      jnp.concatenate([wme, wme], axis=1), _blkdiag(wms), _blkdiag(wmr),
      two(b_msg), _blkdiag(wu1), _blkdiag(wu2), two(b_upd),
      _blkdiag(W_dec), two(b_dec))

    r = out.reshape(B, P, D, 2)
    return r[..., 0], r[..., 1]


# speedup vs baseline: 4.3028x; 1.8714x over previous
import jax
import jax.numpy as jnp
from jax.experimental import pallas as pl
from jax.experimental.pallas import tpu as pltpu

B, P, D, H = 32, 64, 2, 64
G = 4


def _body(na_ref, ea_ref, wn_ref, bn_ref, wet_ref, bet_ref,
          wme_ref, wms_ref, wmr_ref, bm_ref, wu1_ref, wu2_ref, bu_ref,
          wd_ref, bd_ref, out_ref):
    f32 = jnp.float32
    na = na_ref[...].reshape(G * P, D * 5)
    hn = jax.nn.relu(jnp.dot(na, wn_ref[...],
                             preferred_element_type=f32) + bn_ref[...])
    a_term = jnp.dot(hn, wms_ref[...], preferred_element_type=f32)
    c_term = jnp.dot(hn, wmr_ref[...], preferred_element_type=f32)
    hes = []
    for g in range(G):
        het = jax.nn.relu(jnp.dot(wet_ref[...], ea_ref[g],
                                  preferred_element_type=f32) + bet_ref[...])
        hes.append(jnp.transpose(het))
    he = jnp.concatenate(hes, axis=0)
    eh = jnp.dot(he, wme_ref[...], preferred_element_type=f32)
    eh4 = eh.reshape(G, P, P, D * H)
    t = jax.nn.relu(eh4 + a_term.reshape(G, 1, P, D * H)
                    + c_term.reshape(G, P, 1, D * H) + bm_ref[...])
    agg = jnp.sum(t, axis=2).reshape(G * P, D * H)
    h2 = jax.nn.relu(jnp.dot(hn, wu1_ref[...], preferred_element_type=f32)
                     + jnp.dot(agg, wu2_ref[...], preferred_element_type=f32)
                     + bu_ref[...])
    o = jnp.dot(h2, wd_ref[...], preferred_element_type=f32) + bd_ref[...]
    out_ref[...] = o.reshape(G, P, D * 2)


def _blkdiag(w):
    r, c = w.shape
    z = jnp.zeros((r, c), w.dtype)
    return jnp.concatenate(
        [jnp.concatenate([w, z], axis=1), jnp.concatenate([z, w], axis=1)],
        axis=0)


def kernel(dq1, dq2, dp1, dp2, m, t, dt, length, k,
           Wn_enc, bn_enc, We_enc, be_enc, W_msg, b_msg, W_upd, b_upd,
           W_dec, b_dec):
    del t, dt
    m_rep = jnp.tile(m, (1, 1, D))
    na = jnp.stack([dq1, dq2, dp1, dp2, m_rep], axis=-1).reshape(B, P, D * 5)
    ea = jnp.stack([length.reshape(B, P * P), k.reshape(B, P * P)], axis=1)
    wme, wms, wmr = W_msg[:H], W_msg[H:2 * H], W_msg[2 * H:]
    wu1, wu2 = W_upd[:H], W_upd[H:]
    two = lambda v: jnp.concatenate([v, v]).reshape(1, -1)

    per_b3 = lambda shape: pl.BlockSpec(shape, lambda b: (b, 0, 0))
    const2 = lambda shape: pl.BlockSpec(shape, lambda b: (0, 0))

    out = pl.pallas_call(
        _body,
        grid=(B // G,),
        in_specs=[
            per_b3((G, P, D * 5)),
            per_b3((G, 2, P * P)),
            const2((D * 5, D * H)),
            const2((1, D * H)),
            const2((H, 2)),
            const2((H, 1)),
            const2((H, D * H)),
            const2((D * H, D * H)),
            const2((D * H, D * H)),
            const2((1, D * H)),
            const2((D * H, D * H)),
            const2((D * H, D * H)),
            const2((1, D * H)),
            const2((D * H, D * 2)),
            const2((1, D * 2)),
        ],
        out_specs=per_b3((G, P, D * 2)),
        out_shape=jax.ShapeDtypeStruct((B, P, D * 2), jnp.float32),
        compiler_params=pltpu.CompilerParams(
            dimension_semantics=("arbitrary",)),
    )(na, ea, _blkdiag(Wn_enc), two(bn_enc), We_enc.T, be_enc.reshape(-1, 1),
      jnp.concatenate([wme, wme], axis=1), _blkdiag(wms), _blkdiag(wmr),
      two(b_msg), _blkdiag(wu1), _blkdiag(wu2), two(b_upd),
      _blkdiag(W_dec), two(b_dec))

    r = out.reshape(B, P, D, 2)
    return r[..., 0], r[..., 1]


# speedup vs baseline: 4.4966x; 1.0450x over previous
import jax
import jax.numpy as jnp
from jax.experimental import pallas as pl
from jax.experimental.pallas import tpu as pltpu

B, P, D, H = 32, 64, 2, 64
G = 8


def _body(na_ref, ea_ref, wn_ref, bn_ref, wet_ref, bet_ref,
          wme_ref, wms_ref, wmr_ref, bm_ref, wu1_ref, wu2_ref, bu_ref,
          wd_ref, bd_ref, out_ref):
    f32 = jnp.float32
    na = na_ref[...].reshape(G * P, D * 5)
    hn = jax.nn.relu(jnp.dot(na, wn_ref[...],
                             preferred_element_type=f32) + bn_ref[...])
    a_term = jnp.dot(hn, wms_ref[...], preferred_element_type=f32)
    c_term = jnp.dot(hn, wmr_ref[...], preferred_element_type=f32)
    hes = []
    for g in range(G):
        het = jax.nn.relu(jnp.dot(wet_ref[...], ea_ref[g],
                                  preferred_element_type=f32) + bet_ref[...])
        hes.append(jnp.transpose(het))
    he = jnp.concatenate(hes, axis=0)
    eh = jnp.dot(he, wme_ref[...], preferred_element_type=f32)
    eh4 = eh.reshape(G, P, P, D * H)
    t = jax.nn.relu(eh4 + a_term.reshape(G, 1, P, D * H)
                    + c_term.reshape(G, P, 1, D * H) + bm_ref[...])
    agg = jnp.sum(t, axis=2).reshape(G * P, D * H)
    h2 = jax.nn.relu(jnp.dot(hn, wu1_ref[...], preferred_element_type=f32)
                     + jnp.dot(agg, wu2_ref[...], preferred_element_type=f32)
                     + bu_ref[...])
    o = jnp.dot(h2, wd_ref[...], preferred_element_type=f32) + bd_ref[...]
    out_ref[...] = o.reshape(G, P, D * 2)


def _blkdiag(w):
    r, c = w.shape
    z = jnp.zeros((r, c), w.dtype)
    return jnp.concatenate(
        [jnp.concatenate([w, z], axis=1), jnp.concatenate([z, w], axis=1)],
        axis=0)


def kernel(dq1, dq2, dp1, dp2, m, t, dt, length, k,
           Wn_enc, bn_enc, We_enc, be_enc, W_msg, b_msg, W_upd, b_upd,
           W_dec, b_dec):
    del t, dt
    m_rep = jnp.tile(m, (1, 1, D))
    na = jnp.stack([dq1, dq2, dp1, dp2, m_rep], axis=-1).reshape(B, P, D * 5)
    ea = jnp.stack([length.reshape(B, P * P), k.reshape(B, P * P)], axis=1)
    wme, wms, wmr = W_msg[:H], W_msg[H:2 * H], W_msg[2 * H:]
    wu1, wu2 = W_upd[:H], W_upd[H:]
    two = lambda v: jnp.concatenate([v, v]).reshape(1, -1)

    per_b3 = lambda shape: pl.BlockSpec(shape, lambda b: (b, 0, 0))
    const2 = lambda shape: pl.BlockSpec(shape, lambda b: (0, 0))

    out = pl.pallas_call(
        _body,
        grid=(B // G,),
        in_specs=[
            per_b3((G, P, D * 5)),
            per_b3((G, 2, P * P)),
            const2((D * 5, D * H)),
            const2((1, D * H)),
            const2((H, 2)),
            const2((H, 1)),
            const2((H, D * H)),
            const2((D * H, D * H)),
            const2((D * H, D * H)),
            const2((1, D * H)),
            const2((D * H, D * H)),
            const2((D * H, D * H)),
            const2((1, D * H)),
            const2((D * H, D * 2)),
            const2((1, D * 2)),
        ],
        out_specs=per_b3((G, P, D * 2)),
        out_shape=jax.ShapeDtypeStruct((B, P, D * 2), jnp.float32),
        compiler_params=pltpu.CompilerParams(
            dimension_semantics=("arbitrary",)),
    )(na, ea, _blkdiag(Wn_enc), two(bn_enc), We_enc.T, be_enc.reshape(-1, 1),
      jnp.concatenate([wme, wme], axis=1), _blkdiag(wms), _blkdiag(wmr),
      two(b_msg), _blkdiag(wu1), _blkdiag(wu2), two(b_upd),
      _blkdiag(W_dec), two(b_dec))

    r = out.reshape(B, P, D, 2)
    return r[..., 0], r[..., 1]
